# baseline (speedup 1.0000x reference)
import functools

import jax
import jax.numpy as jnp
import numpy as np
from jax.experimental import pallas as pl
from jax.experimental.pallas import tpu as pltpu

FACTOR = 5
NEG_INF = float("-inf")

_COUNTS_CACHE = {}


def _counts_t(L, L_K, U_part):
    ck = (L, L_K, U_part)
    if ck not in _COUNTS_CACHE:
        with jax.ensure_compile_time_eval():
            idx = np.asarray(jax.random.randint(
                jax.random.key(42), (L, U_part), 0, L_K))
        ct = np.zeros((L_K, L), np.int8)
        np.add.at(ct, (idx.ravel(),
                       np.repeat(np.arange(L), U_part)), 1)
        _COUNTS_CACHE[ck] = ct
    return _COUNTS_CACHE[ck]


def _head_kernel(i_ref, q_ref, k_ref, v_ref, o_ref,
                 m_ref, oh_ref,
                 *, u, blk_q, blk_c, d_head):
    L = q_ref.shape[0]
    D = d_head
    n_heads = q_ref.shape[1] // d_head
    L_K = k_ref.shape[0]
    scale = 1.0 / np.sqrt(D)

    tri = (jax.lax.broadcasted_iota(jnp.int32, (blk_c, blk_c), 0)
           >= jax.lax.broadcasted_iota(jnp.int32, (blk_c, blk_c), 1)
           ).astype(jnp.float32)
    n_cblk = L // blk_c
    carry = jnp.zeros((1, q_ref.shape[1]), jnp.float32)
    for b in range(n_cblk):
        rows = pl.ds(b * blk_c, blk_c)
        blk = jax.lax.dot_general(
            tri, v_ref[rows, :],
            (((1,), (0,)), ((), ())),
            preferred_element_type=jnp.float32,
            precision=jax.lax.Precision.HIGHEST) + carry
        o_ref[rows, :] = blk
        carry = blk[blk_c - 1:blk_c, :]

    lin = jax.lax.broadcasted_iota(jnp.int32, (1, L), 1)
    ri = jax.lax.broadcasted_iota(jnp.int32, (u, u), 0)
    ci = jax.lax.broadcasted_iota(jnp.int32, (u, u), 1)
    n_blk = L // blk_q
    for hh in range(n_heads):
        cols = slice(hh * D, (hh + 1) * D)

        for b in range(n_blk):
            rows = pl.ds(b * blk_q, blk_q)
            s_t = jax.lax.dot_general(
                k_ref[:, cols], q_ref[rows, cols],
                (((1,), (1,)), ((), ())),
                preferred_element_type=jnp.float32,
                precision=jax.lax.Precision.DEFAULT)
            cnt = i_ref[:, rows].astype(jnp.float32)
            mx = jnp.max(jnp.where(cnt > 0.0, s_t, NEG_INF), axis=0)
            sm = jnp.sum(s_t * cnt, axis=0)
            m_ref[:, rows] = (mx - sm * (1.0 / L_K))[None, :]

        def topk_body(i, mv):
            mmax = jnp.max(mv, keepdims=True)
            jb = jnp.min(jnp.where(mv == mmax, lin, L), keepdims=True)
            sel = lin == jb
            oh_ref[pl.ds(i, 1), :] = sel.astype(jnp.float32)
            return jnp.where(sel, NEG_INF, mv)

        jax.lax.fori_loop(0, u, topk_body, m_ref[...], unroll=False)

        qr = jax.lax.dot_general(
            oh_ref[...], q_ref[:, cols],
            (((1,), (0,)), ((), ())),
            preferred_element_type=jnp.float32,
            precision=jax.lax.Precision.HIGHEST)
        s2 = jax.lax.dot_general(
            qr, k_ref[0:u, cols],
            (((1,), (1,)), ((), ())),
            preferred_element_type=jnp.float32,
            precision=jax.lax.Precision.DEFAULT) * scale
        s2 = jnp.where(ci > ri, NEG_INF, s2)
        s2 = s2 - jnp.max(s2, axis=1, keepdims=True)
        e = jnp.exp(s2)
        attn = e / jnp.sum(e, axis=1, keepdims=True)
        upd = jax.lax.dot_general(
            attn, v_ref[0:u, cols],
            (((1,), (0,)), ((), ())),
            preferred_element_type=jnp.float32,
            precision=jax.lax.Precision.HIGHEST)

        aug = jnp.concatenate(
            [upd, jnp.ones((u, 1), jnp.float32),
             jnp.zeros((u, D - 1), jnp.float32)], axis=1)
        res = jax.lax.dot_general(
            oh_ref[...], aug,
            (((0,), (0,)), ((), ())),
            preferred_element_type=jnp.float32,
            precision=jax.lax.Precision.HIGHEST)
        o_ref[:, cols] = jnp.where(
            res[:, D:D + 1] > 0.0, res[:, 0:D], o_ref[:, cols])


def kernel(queries, keys, values):
    B, L, H, D = queries.shape
    L_K = keys.shape[1]
    U_part = min(int(FACTOR * np.ceil(np.log(L_K))), L_K)
    u = min(int(FACTOR * np.ceil(np.log(L))), L)
    assert U_part == u

    counts_t = jnp.asarray(_counts_t(L, L_K, U_part))

    qf = queries.reshape(B, L, H * D)
    kf = keys.reshape(B, L, H * D)
    vf = values.reshape(B, L, H * D)

    blk_q, blk_c = 512, 256
    hp = 128 // D
    out = pl.pallas_call(
        functools.partial(_head_kernel, u=u, blk_q=blk_q, blk_c=blk_c,
                          d_head=D),
        grid=(B, H // hp),
        in_specs=[
            pl.BlockSpec((L_K, L), lambda b, h: (0, 0)),
            pl.BlockSpec((None, L, hp * D), lambda b, h: (b, 0, h)),
            pl.BlockSpec((None, L, hp * D), lambda b, h: (b, 0, h)),
            pl.BlockSpec((None, L, hp * D), lambda b, h: (b, 0, h)),
        ],
        out_specs=pl.BlockSpec((None, L, hp * D), lambda b, h: (b, 0, h)),
        out_shape=jax.ShapeDtypeStruct((B, L, H * D), jnp.float32),
        scratch_shapes=[
            pltpu.VMEM((1, L), jnp.float32),
            pltpu.VMEM((u, L), jnp.float32),
        ],
    )(counts_t, qf, kf, vf)

    return out.reshape(B, L, H, D)


# speedup vs baseline: 1.0338x; 1.0338x over previous
import functools

import jax
import jax.numpy as jnp
import numpy as np
from jax.experimental import pallas as pl
from jax.experimental.pallas import tpu as pltpu

FACTOR = 5
NEG_INF = float("-inf")

_COUNTS_CACHE = {}


def _counts_t(L, L_K, U_part):
    ck = (L, L_K, U_part)
    if ck not in _COUNTS_CACHE:
        with jax.ensure_compile_time_eval():
            idx = np.asarray(jax.random.randint(
                jax.random.key(42), (L, U_part), 0, L_K))
        ct = np.zeros((L_K, L), np.float32)
        np.add.at(ct, (idx.ravel(),
                       np.repeat(np.arange(L), U_part)), 1.0)
        _COUNTS_CACHE[ck] = ct
    return _COUNTS_CACHE[ck]


def _head_kernel(i_ref, q_ref, k_ref, v_ref, o_ref,
                 m_ref, oh_ref,
                 *, u, blk_q, blk_c, d_head):
    L = q_ref.shape[0]
    D = d_head
    n_heads = q_ref.shape[1] // d_head
    L_K = k_ref.shape[0]
    scale = 1.0 / np.sqrt(D)

    tri = (jax.lax.broadcasted_iota(jnp.int32, (blk_c, blk_c), 0)
           >= jax.lax.broadcasted_iota(jnp.int32, (blk_c, blk_c), 1)
           ).astype(jnp.float32)
    n_cblk = L // blk_c
    carry = jnp.zeros((1, q_ref.shape[1]), jnp.float32)
    for b in range(n_cblk):
        rows = pl.ds(b * blk_c, blk_c)
        blk = jax.lax.dot_general(
            tri, v_ref[rows, :],
            (((1,), (0,)), ((), ())),
            preferred_element_type=jnp.float32,
            precision=jax.lax.Precision.HIGHEST) + carry
        o_ref[rows, :] = blk
        carry = blk[blk_c - 1:blk_c, :]

    lin = jax.lax.broadcasted_iota(jnp.int32, (1, L), 1)
    ri = jax.lax.broadcasted_iota(jnp.int32, (u, u), 0)
    ci = jax.lax.broadcasted_iota(jnp.int32, (u, u), 1)
    n_blk = L // blk_q
    for hh in range(n_heads):
        cols = slice(hh * D, (hh + 1) * D)

        for b in range(n_blk):
            rows = pl.ds(b * blk_q, blk_q)
            s_t = jax.lax.dot_general(
                k_ref[:, cols], q_ref[rows, cols],
                (((1,), (1,)), ((), ())),
                preferred_element_type=jnp.float32,
                precision=jax.lax.Precision.DEFAULT)
            cnt = i_ref[:, rows]
            mx = jnp.max(jnp.where(cnt > 0.0, s_t, NEG_INF), axis=0)
            sm = jnp.sum(s_t * cnt, axis=0)
            m_ref[:, rows] = (mx - sm * (1.0 / L_K))[None, :]

        def topk_body(i, mv):
            mmax = jnp.max(mv, keepdims=True)
            jb = jnp.min(jnp.where(mv == mmax, lin, L), keepdims=True)
            sel = lin == jb
            oh_ref[pl.ds(i, 1), :] = sel.astype(jnp.float32)
            return jnp.where(sel, NEG_INF, mv)

        jax.lax.fori_loop(0, u, topk_body, m_ref[...], unroll=False)

        qr = jax.lax.dot_general(
            oh_ref[...], q_ref[:, cols],
            (((1,), (0,)), ((), ())),
            preferred_element_type=jnp.float32,
            precision=jax.lax.Precision.HIGHEST)
        s2 = jax.lax.dot_general(
            qr, k_ref[0:u, cols],
            (((1,), (1,)), ((), ())),
            preferred_element_type=jnp.float32,
            precision=jax.lax.Precision.DEFAULT) * scale
        s2 = jnp.where(ci > ri, NEG_INF, s2)
        s2 = s2 - jnp.max(s2, axis=1, keepdims=True)
        e = jnp.exp(s2)
        attn = e / jnp.sum(e, axis=1, keepdims=True)
        upd = jax.lax.dot_general(
            attn, v_ref[0:u, cols],
            (((1,), (0,)), ((), ())),
            preferred_element_type=jnp.float32,
            precision=jax.lax.Precision.HIGHEST)

        aug = jnp.concatenate(
            [upd, jnp.ones((u, 1), jnp.float32),
             jnp.zeros((u, D - 1), jnp.float32)], axis=1)
        res = jax.lax.dot_general(
            oh_ref[...], aug,
            (((0,), (0,)), ((), ())),
            preferred_element_type=jnp.float32,
            precision=jax.lax.Precision.HIGHEST)
        o_ref[:, cols] = jnp.where(
            res[:, D:D + 1] > 0.0, res[:, 0:D], o_ref[:, cols])


def kernel(queries, keys, values):
    B, L, H, D = queries.shape
    L_K = keys.shape[1]
    U_part = min(int(FACTOR * np.ceil(np.log(L_K))), L_K)
    u = min(int(FACTOR * np.ceil(np.log(L))), L)
    assert U_part == u

    counts_t = jnp.asarray(_counts_t(L, L_K, U_part))

    qf = queries.reshape(B, L, H * D)
    kf = keys.reshape(B, L, H * D)
    vf = values.reshape(B, L, H * D)

    blk_q, blk_c = 1024, 256
    hp = 128 // D
    out = pl.pallas_call(
        functools.partial(_head_kernel, u=u, blk_q=blk_q, blk_c=blk_c,
                          d_head=D),
        grid=(B, H // hp),
        in_specs=[
            pl.BlockSpec((L_K, L), lambda b, h: (0, 0)),
            pl.BlockSpec((None, L, hp * D), lambda b, h: (b, 0, h)),
            pl.BlockSpec((None, L, hp * D), lambda b, h: (b, 0, h)),
            pl.BlockSpec((None, L, hp * D), lambda b, h: (b, 0, h)),
        ],
        out_specs=pl.BlockSpec((None, L, hp * D), lambda b, h: (b, 0, h)),
        out_shape=jax.ShapeDtypeStruct((B, L, H * D), jnp.float32),
        scratch_shapes=[
            pltpu.VMEM((1, L), jnp.float32),
            pltpu.VMEM((u, L), jnp.float32),
        ],
    )(counts_t, qf, kf, vf)

    return out.reshape(B, L, H, D)
